# baseline (speedup 1.0000x reference)
import jax
import jax.numpy as jnp
from jax.experimental import pallas as pl
from jax.experimental.pallas import tpu as pltpu


_G = 2


def _relu(v):
    return jnp.maximum(v, 0.0)


def _gin_kernel(x_ref, a_hbm, w1a_ref, b1a_ref, w1b_ref, b1b_ref,
                w2a_ref, b2a_ref, w2b_ref, b2b_ref, g_ref, a_buf, sems):
    b = pl.program_id(0)
    nsteps = pl.num_programs(0)
    slot = jax.lax.rem(b, 2)
    nxt = jax.lax.rem(b + 1, 2)

    def copies(step, buf_slot):
        return [pltpu.make_async_copy(a_hbm.at[step * _G + i],
                                      a_buf.at[buf_slot, i],
                                      sems.at[buf_slot, i])
                for i in range(_G)]

    @pl.when(b == 0)
    def _():
        for c in copies(0, 0):
            c.start()

    @pl.when(b + 1 < nsteps)
    def _():
        for c in copies(b + 1, nxt):
            c.start()

    for c in copies(b, slot):
        c.wait()

    dot = lambda p, q: jnp.dot(p, q, preferred_element_type=jnp.float32)
    dot_t = lambda p, q: jax.lax.dot_general(
        p, q, (((1,), (1,)), ((), ())), preferred_element_type=jnp.float32)
    a = [a_buf[slot, i] for i in range(_G)]
    xT = [x_ref[i] for i in range(_G)]

    m = [dot_t(xT[i], a[i]) + xT[i] for i in range(_G)]
    m = [_relu(dot(w1a_ref[...], v) + b1a_ref[...]) for v in m]
    h1 = [_relu(dot(w1b_ref[...], v) + b1b_ref[...]) for v in m]
    m2 = [dot_t(h1[i], a[i]) + h1[i] for i in range(_G)]
    m2 = [_relu(dot(w2a_ref[...], v) + b2a_ref[...]) for v in m2]
    h2 = [_relu(dot(w2b_ref[...], v) + b2b_ref[...]) for v in m2]
    for i in range(_G):
        g_ref[i] = jnp.sum(h2[i], axis=1, keepdims=True).T


def _head_kernel(g_ref, u_ref, ln_g_ref, ln_b_ref, wf1_ref, bf1_ref,
                 wf2_ref, bf2_ref, wv1_ref, bv1_ref, wv2_ref, bv2_ref,
                 wa1_ref, ba1_ref, wa2_ref, ba2_ref, out_ref):
    z = jnp.concatenate([g_ref[...], u_ref[...]], axis=1)

    mu = jnp.mean(z, axis=1, keepdims=True)
    var = jnp.mean((z - mu) ** 2, axis=1, keepdims=True)
    z = (z - mu) * jax.lax.rsqrt(var + 1e-3) * ln_g_ref[...] + ln_b_ref[...]

    z = _relu(jnp.dot(z, wf1_ref[...], preferred_element_type=jnp.float32)
              + bf1_ref[...])
    z = _relu(jnp.dot(z, wf2_ref[...], preferred_element_type=jnp.float32)
              + bf2_ref[...])

    v = jnp.dot(_relu(jnp.dot(z, wv1_ref[...],
                              preferred_element_type=jnp.float32)
                      + bv1_ref[...]),
                wv2_ref[...], preferred_element_type=jnp.float32) + bv2_ref[...]
    ast = jnp.dot(_relu(jnp.dot(z, wa1_ref[...],
                                preferred_element_type=jnp.float32)
                        + ba1_ref[...]),
                  wa2_ref[...], preferred_element_type=jnp.float32) + ba2_ref[...]
    ast = ast - jnp.mean(ast, axis=1, keepdims=True)
    out_ref[...] = v + ast


@jax.jit
def kernel(x, a, u, w1a, b1a, w1b, b1b, w2a, b2a, w2b, b2b, ln_g, ln_b,
           wf1, bf1, wf2, bf2, wv1, bv1, wv2, bv2, wa1, ba1, wa2, ba2):
    B, N, F = x.shape
    H = w1b.shape[1]
    U = u.shape[1]
    A_DIM = wa2.shape[1]

    row = lambda v: v.reshape(1, -1)
    col = lambda v: v.reshape(-1, 1)
    ln_g, ln_b = row(ln_g), row(ln_b)
    bf1, bf2, bv1, bv2, ba1, ba2 = (row(bf1), row(bf2), row(bv1), row(bv2),
                                    row(ba1), row(ba2))
    w1aT, w1bT, w2aT, w2bT = w1a.T, w1b.T, w2a.T, w2b.T
    b1a, b1b, b2a, b2b = col(b1a), col(b1b), col(b2a), col(b2b)
    xT = x.transpose(0, 2, 1)

    full = lambda arr: pl.BlockSpec(arr.shape, lambda b: (0,) * arr.ndim)
    gin_grid = (B // _G,)
    gin_in_specs = [
            pl.BlockSpec((_G, F, N), lambda b: (b, 0, 0)),
            pl.BlockSpec(memory_space=pltpu.MemorySpace.HBM),
            full(w1aT), full(b1a), full(w1bT), full(b1b),
            full(w2aT), full(b2a), full(w2bT), full(b2b),
        ]
    g = pl.pallas_call(
        _gin_kernel,
        grid=gin_grid,
        in_specs=gin_in_specs,
        out_specs=pl.BlockSpec((_G, 1, H), lambda b: (b, 0, 0)),
        out_shape=jax.ShapeDtypeStruct((B, 1, H), jnp.float32),
        scratch_shapes=[
            pltpu.VMEM((2, _G, N, N), jnp.float32),
            pltpu.SemaphoreType.DMA((2, _G)),
        ],
        compiler_params=pltpu.CompilerParams(
            dimension_semantics=("arbitrary",),
        ),
    )(xT, a, w1aT, b1a, w1bT, b1b, w2aT, b2a, w2bT, b2b)
    g = g.reshape(B, H)

    head_in = [g, u, ln_g, ln_b, wf1, bf1, wf2, bf2,
               wv1, bv1, wv2, bv2, wa1, ba1, wa2, ba2]
    whole = lambda arr: pl.BlockSpec(arr.shape, lambda: (0,) * arr.ndim)
    return pl.pallas_call(
        _head_kernel,
        in_specs=[whole(arr) for arr in head_in],
        out_specs=pl.BlockSpec((B, A_DIM), lambda: (0, 0)),
        out_shape=jax.ShapeDtypeStruct((B, A_DIM), jnp.float32),
    )(*head_in)


# speedup vs baseline: 1.1036x; 1.1036x over previous
import jax
import jax.numpy as jnp
from jax.experimental import pallas as pl
from jax.experimental.pallas import tpu as pltpu


_G = 2


def _relu(v):
    return jnp.maximum(v, 0.0)


def _gin_kernel(x_ref, a_ref, w1a_ref, b1a_ref, w1b_ref, b1b_ref,
                w2a_ref, b2a_ref, w2b_ref, b2b_ref, g_ref):
    dot = lambda p, q: jnp.dot(p, q, preferred_element_type=jnp.float32)

    def agg(a_full, h):
        n = a_full.shape[1]
        lo = dot(a_full[:, : n // 2], h[: n // 2])
        hi = dot(a_full[:, n // 2:], h[n // 2:])
        return lo + hi

    a = [a_ref[i] for i in range(_G)]
    x = [x_ref[i] for i in range(_G)]

    m = [agg(a[i], x[i]) + x[i] for i in range(_G)]
    m = [_relu(dot(v, w1a_ref[...]) + b1a_ref[...]) for v in m]
    h1 = [_relu(dot(v, w1b_ref[...]) + b1b_ref[...]) for v in m]
    m2 = [agg(a[i], h1[i]) + h1[i] for i in range(_G)]
    m2 = [_relu(dot(v, w2a_ref[...]) + b2a_ref[...]) for v in m2]
    h2 = [_relu(dot(v, w2b_ref[...]) + b2b_ref[...]) for v in m2]
    for i in range(_G):
        g_ref[i] = jnp.sum(h2[i], axis=0, keepdims=True)


def _head_kernel(g_ref, u_ref, ln_g_ref, ln_b_ref, wf1_ref, bf1_ref,
                 wf2_ref, bf2_ref, wv1_ref, bv1_ref, wv2_ref, bv2_ref,
                 wa1_ref, ba1_ref, wa2_ref, ba2_ref, out_ref):
    z = jnp.concatenate([g_ref[...], u_ref[...]], axis=1)

    mu = jnp.mean(z, axis=1, keepdims=True)
    var = jnp.mean((z - mu) ** 2, axis=1, keepdims=True)
    z = (z - mu) * jax.lax.rsqrt(var + 1e-3) * ln_g_ref[...] + ln_b_ref[...]

    z = _relu(jnp.dot(z, wf1_ref[...], preferred_element_type=jnp.float32)
              + bf1_ref[...])
    z = _relu(jnp.dot(z, wf2_ref[...], preferred_element_type=jnp.float32)
              + bf2_ref[...])

    v = jnp.dot(_relu(jnp.dot(z, wv1_ref[...],
                              preferred_element_type=jnp.float32)
                      + bv1_ref[...]),
                wv2_ref[...], preferred_element_type=jnp.float32) + bv2_ref[...]
    ast = jnp.dot(_relu(jnp.dot(z, wa1_ref[...],
                                preferred_element_type=jnp.float32)
                        + ba1_ref[...]),
                  wa2_ref[...], preferred_element_type=jnp.float32) + ba2_ref[...]
    ast = ast - jnp.mean(ast, axis=1, keepdims=True)
    out_ref[...] = v + ast


@jax.jit
def kernel(x, a, u, w1a, b1a, w1b, b1b, w2a, b2a, w2b, b2b, ln_g, ln_b,
           wf1, bf1, wf2, bf2, wv1, bv1, wv2, bv2, wa1, ba1, wa2, ba2):
    B, N, F = x.shape
    H = w1b.shape[1]
    U = u.shape[1]
    A_DIM = wa2.shape[1]

    row = lambda v: v.reshape(1, -1)
    b1a, b1b, b2a, b2b = row(b1a), row(b1b), row(b2a), row(b2b)
    ln_g, ln_b = row(ln_g), row(ln_b)
    bf1, bf2, bv1, bv2, ba1, ba2 = (row(bf1), row(bf2), row(bv1), row(bv2),
                                    row(ba1), row(ba2))

    full = lambda arr: pl.BlockSpec(arr.shape, lambda b: (0,) * arr.ndim)
    g = pl.pallas_call(
        _gin_kernel,
        grid=(B // _G,),
        in_specs=[
            pl.BlockSpec((_G, N, F), lambda b: (b, 0, 0)),
            pl.BlockSpec((_G, N, N), lambda b: (b, 0, 0)),
            full(w1a), full(b1a), full(w1b), full(b1b),
            full(w2a), full(b2a), full(w2b), full(b2b),
        ],
        out_specs=pl.BlockSpec((_G, 1, H), lambda b: (b, 0, 0)),
        out_shape=jax.ShapeDtypeStruct((B, 1, H), jnp.float32),
        compiler_params=pltpu.CompilerParams(
            dimension_semantics=("parallel",),
        ),
    )(x, a, w1a, b1a, w1b, b1b, w2a, b2a, w2b, b2b)
    g = g.reshape(B, H)

    head_in = [g, u, ln_g, ln_b, wf1, bf1, wf2, bf2,
               wv1, bv1, wv2, bv2, wa1, ba1, wa2, ba2]
    whole = lambda arr: pl.BlockSpec(arr.shape, lambda: (0,) * arr.ndim)
    return pl.pallas_call(
        _head_kernel,
        in_specs=[whole(arr) for arr in head_in],
        out_specs=pl.BlockSpec((B, A_DIM), lambda: (0, 0)),
        out_shape=jax.ShapeDtypeStruct((B, A_DIM), jnp.float32),
    )(*head_in)


# speedup vs baseline: 1.1735x; 1.0634x over previous
import jax
import jax.numpy as jnp
from jax.experimental import pallas as pl
from jax.experimental.pallas import tpu as pltpu


_G = 2


def _relu(v):
    return jnp.maximum(v, 0.0)


def _fused_kernel(x_ref, a_ref, u_ref, w1a_ref, b1a_ref, w1b_ref, b1b_ref,
                  w2a_ref, b2a_ref, w2b_ref, b2b_ref, ln_g_ref, ln_b_ref,
                  wf1_ref, bf1_ref, wf2_ref, bf2_ref, wv1_ref, bv1_ref,
                  wv2_ref, bv2_ref, wa1_ref, ba1_ref, wa2_ref, ba2_ref,
                  out_ref, g_scr):
    b = pl.program_id(0)
    nsteps = pl.num_programs(0)
    dot = lambda p, q: jnp.dot(p, q, preferred_element_type=jnp.float32)

    a = [a_ref[i] for i in range(_G)]
    x = [x_ref[i] for i in range(_G)]

    m = [dot(a[i], x[i]) + x[i] for i in range(_G)]
    m = [_relu(dot(v, w1a_ref[...]) + b1a_ref[...]) for v in m]
    h1 = [_relu(dot(v, w1b_ref[...]) + b1b_ref[...]) for v in m]
    m2 = [dot(a[i], h1[i]) + h1[i] for i in range(_G)]
    m2 = [_relu(dot(v, w2a_ref[...]) + b2a_ref[...]) for v in m2]
    h2 = [_relu(dot(v, w2b_ref[...]) + b2b_ref[...]) for v in m2]
    for i in range(_G):
        g_scr[pl.ds(b * _G + i, 1), :] = jnp.sum(h2[i], axis=0, keepdims=True)

    @pl.when(b == nsteps - 1)
    def _():
        z = jnp.concatenate([g_scr[...], u_ref[...]], axis=1)

        mu = jnp.mean(z, axis=1, keepdims=True)
        var = jnp.mean((z - mu) ** 2, axis=1, keepdims=True)
        zn = (z - mu) * jax.lax.rsqrt(var + 1e-3) * ln_g_ref[...] + ln_b_ref[...]

        t = _relu(dot(zn, wf1_ref[...]) + bf1_ref[...])
        t = _relu(dot(t, wf2_ref[...]) + bf2_ref[...])

        v = dot(_relu(dot(t, wv1_ref[...]) + bv1_ref[...]),
                wv2_ref[...]) + bv2_ref[...]
        ast = dot(_relu(dot(t, wa1_ref[...]) + ba1_ref[...]),
                  wa2_ref[...]) + ba2_ref[...]
        ast = ast - jnp.mean(ast, axis=1, keepdims=True)
        out_ref[...] = v + ast


@jax.jit
def kernel(x, a, u, w1a, b1a, w1b, b1b, w2a, b2a, w2b, b2b, ln_g, ln_b,
           wf1, bf1, wf2, bf2, wv1, bv1, wv2, bv2, wa1, ba1, wa2, ba2):
    B, N, F = x.shape
    H = w1b.shape[1]
    U = u.shape[1]
    A_DIM = wa2.shape[1]

    row = lambda v: v.reshape(1, -1)
    b1a, b1b, b2a, b2b = row(b1a), row(b1b), row(b2a), row(b2b)
    ln_g, ln_b = row(ln_g), row(ln_b)
    bf1, bf2, bv1, bv2, ba1, ba2 = (row(bf1), row(bf2), row(bv1), row(bv2),
                                    row(ba1), row(ba2))

    full = lambda arr: pl.BlockSpec(arr.shape, lambda b: (0,) * arr.ndim)
    return pl.pallas_call(
        _fused_kernel,
        grid=(B // _G,),
        in_specs=[
            pl.BlockSpec((_G, N, F), lambda b: (b, 0, 0)),
            pl.BlockSpec((_G, N, N), lambda b: (b, 0, 0)),
            full(u),
            full(w1a), full(b1a), full(w1b), full(b1b),
            full(w2a), full(b2a), full(w2b), full(b2b),
            full(ln_g), full(ln_b),
            full(wf1), full(bf1), full(wf2), full(bf2),
            full(wv1), full(bv1), full(wv2), full(bv2),
            full(wa1), full(ba1), full(wa2), full(ba2),
        ],
        out_specs=pl.BlockSpec((B, A_DIM), lambda b: (0, 0)),
        out_shape=jax.ShapeDtypeStruct((B, A_DIM), jnp.float32),
        scratch_shapes=[pltpu.VMEM((B, H), jnp.float32)],
        compiler_params=pltpu.CompilerParams(
            dimension_semantics=("arbitrary",),
        ),
    )(x, a, u, w1a, b1a, w1b, b1b, w2a, b2a, w2b, b2b, ln_g, ln_b,
      wf1, bf1, wf2, bf2, wv1, bv1, wv2, bv2, wa1, ba1, wa2, ba2)
